# baseline (speedup 1.0000x reference)
import math

import jax
import jax.numpy as jnp
from jax import lax
from jax.experimental import pallas as pl
from jax.experimental.pallas import tpu as pltpu
from jax.experimental.pallas import tpu_sc as plsc

N = 1048576
D = 16
K = 16
L = 16
NC = 2
NS = 16
NW = NC * NS
NSC = 655360
PTS_PER_W = NSC // NW
CHUNK = 2048
NCHUNK = PTS_PER_W // CHUNK
GROUPS = CHUNK // L
NB = N // 128
CB = CHUNK // 128
BPW = PTS_PER_W // 128
TCB = 4096
NTC = (N - NSC) // TCB
LOG2PI = math.log(2.0 * math.pi)


def _sc_body(xs_hbm, ids_hbm, g_out, c_out, s_out,
             xa, xb, ia, ib, gacc, cacc, gsm, csm, sqv,
             sxa, sxb, sia, sib):
    wid = lax.axis_index("s") * NC + lax.axis_index("c")
    pbase0 = wid * PTS_PER_W

    zero = jnp.zeros((L,), jnp.float32)
    for k in range(K * D):
        gacc[pl.ds(k * L, L)] = zero
    for k in range(K):
        cacc[pl.ds(k * L, L)] = zero

    iota = lax.iota(jnp.int32, L)
    ones = jnp.ones((L,), jnp.float32)

    def xcopy0(c, buf, sem):
        return pltpu.make_async_copy(
            xs_hbm.at[0, pl.ds(wid * BPW + c * CB, CB), :, :], buf.at[0], sem)

    def xcopy1(c, buf, sem):
        return pltpu.make_async_copy(
            xs_hbm.at[1, pl.ds(wid * BPW + c * CB, CB), :, :], buf.at[1], sem)

    def xcopy(c, buf, sem):
        class _Pair:
            def start(self):
                xcopy0(c, buf, sem).start()
                xcopy1(c, buf, sem).start()

            def wait(self):
                xcopy0(c, buf, sem).wait()
                xcopy1(c, buf, sem).wait()
        return _Pair()

    def icopy(c, buf, sem):
        return pltpu.make_async_copy(
            ids_hbm.at[pl.ds(pbase0 + c * CHUNK, CHUNK)], buf, sem)

    def process(xbuf, ibuf, sq):
        def grp(g, sq):
            sq0, sq1, sq2, sq3 = sq
            idv = ibuf[pl.ds(g * L, L)]
            pb = idv * L + iota
            plsc.addupdate_scatter(cacc, [pb], ones)
            jb = g // 8
            ii0 = (g % 8) * L
            xc = [xbuf[dt, jb, dr, pl.ds(ii0, L)]
                  for dt in range(2) for dr in range(8)]
            for d in range(D):
                plsc.addupdate_scatter(
                    gacc.at[pl.ds(d * K * L, K * L)], [pb], xc[d])
            for d in range(0, D, 4):
                sq0 = sq0 + xc[d] * xc[d]
                sq1 = sq1 + xc[d + 1] * xc[d + 1]
                sq2 = sq2 + xc[d + 2] * xc[d + 2]
                sq3 = sq3 + xc[d + 3] * xc[d + 3]
            return sq0, sq1, sq2, sq3
        return plsc.parallel_loop(0, GROUPS, 1, unroll=2, carry=sq)(grp)

    xcopy(0, xa, sxa).start()
    icopy(0, ia, sia).start()

    def outer(i, sq):
        ca = 2 * i
        cb = 2 * i + 1
        xcopy(ca, xa, sxa).wait()
        icopy(ca, ia, sia).wait()
        xcopy(cb, xb, sxb).start()
        icopy(cb, ib, sib).start()
        sq = process(xa, ia, sq)
        xcopy(cb, xb, sxb).wait()
        icopy(cb, ib, sib).wait()

        @pl.when(cb + 1 < NCHUNK)
        def _():
            xcopy(cb + 1, xa, sxa).start()
            icopy(cb + 1, ia, sia).start()

        sq = process(xb, ib, sq)
        return sq

    zacc = (jnp.zeros((L,), jnp.float32),) * 4
    sq = lax.fori_loop(0, NCHUNK // 2, outer, zacc)

    sqv[...] = sq[0] + sq[1] + sq[2] + sq[3]

    mask15 = iota == jnp.full((L,), L - 1, jnp.int32)
    for m in range(K * D):
        v = gacc[pl.ds(m * L, L)]
        plsc.store_scatter(gsm, [jnp.full((L,), m, jnp.int32)],
                           plsc.cumsum(v), mask=mask15)
    for m in range(K):
        v = cacc[pl.ds(m * L, L)]
        plsc.store_scatter(csm, [jnp.full((L,), m, jnp.int32)],
                           plsc.cumsum(v), mask=mask15)

    pltpu.sync_copy(gsm, g_out.at[wid])
    pltpu.sync_copy(csm, c_out.at[wid])
    pltpu.sync_copy(sqv, s_out.at[wid])


_sc_pass = pl.kernel(
    _sc_body,
    out_type=(
        jax.ShapeDtypeStruct((NW, K * D), jnp.float32),
        jax.ShapeDtypeStruct((NW, K), jnp.float32),
        jax.ShapeDtypeStruct((NW, L), jnp.float32),
    ),
    mesh=plsc.VectorSubcoreMesh(core_axis_name="c", subcore_axis_name="s"),
    compiler_params=pltpu.CompilerParams(
        needs_layout_passes=False, use_tc_tiling_on_sc=False),
    scratch_types=[
        pltpu.VMEM((2, CB, 8, 128), jnp.float32),
        pltpu.VMEM((2, CB, 8, 128), jnp.float32),
        pltpu.VMEM((CHUNK,), jnp.int32),
        pltpu.VMEM((CHUNK,), jnp.int32),
        pltpu.VMEM((L * K * D,), jnp.float32),
        pltpu.VMEM((L * K,), jnp.float32),
        pltpu.VMEM((K * D,), jnp.float32),
        pltpu.VMEM((K,), jnp.float32),
        pltpu.VMEM((L,), jnp.float32),
        pltpu.SemaphoreType.DMA,
        pltpu.SemaphoreType.DMA,
        pltpu.SemaphoreType.DMA,
        pltpu.SemaphoreType.DMA,
    ],
)


def _tc_body(x_ref, i_ref, g_ref, aux_ref):
    b = pl.program_id(0)

    @pl.when(b == 0)
    def _():
        g_ref[...] = jnp.zeros_like(g_ref)
        aux_ref[...] = jnp.zeros_like(aux_ref)

    x = x_ref[...]
    ids = i_ref[0, 0, :]
    oh = (ids[:, None]
          == lax.broadcasted_iota(jnp.int32, (TCB, K), 1)
          ).astype(jnp.float32)
    g_ref[...] += lax.dot_general(
        oh, x, (((0,), (0,)), ((), ())),
        preferred_element_type=jnp.float32)
    aux_ref[...] += jnp.stack(
        [jnp.sum(oh, axis=0), jnp.sum(x * x, axis=0)])


_tc_pass = pl.pallas_call(
    _tc_body,
    grid=(NTC,),
    in_specs=[
        pl.BlockSpec((TCB, D), lambda b: (b + NSC // TCB, 0)),
        pl.BlockSpec((1, 1, TCB), lambda b: (b + NSC // TCB, 0, 0)),
    ],
    out_specs=[
        pl.BlockSpec((K, D), lambda b: (0, 0)),
        pl.BlockSpec((2, K), lambda b: (0, 0)),
    ],
    out_shape=[
        jax.ShapeDtypeStruct((K, D), jnp.float32),
        jax.ShapeDtypeStruct((2, K), jnp.float32),
    ],
)


def _combine_body(g_ref, c_ref, s_ref, gt_ref, aux_ref, mu_ref, muf_ref,
                  mm_ref, bp_ref, o_ref):
    g = jnp.sum(g_ref[...], axis=0)
    aux = aux_ref[...]
    cnt = jnp.sum(c_ref[...], axis=0) + aux[0]
    s_total = jnp.sum(s_ref[...]) + jnp.sum(aux[1])
    mu = mu_ref[...]
    musq = jnp.sum(mu * mu, axis=1)
    logbp = jnp.log(bp_ref[...])[0]
    dot = (jnp.sum(g * muf_ref[...][0])
           + jnp.sum(gt_ref[...] * mu))
    w_term = jnp.sum(cnt * (logbp - 0.5 * musq))
    pm = mu - mm_ref[...]
    prior = -0.5 * jnp.sum(pm * pm) - K * (0.5 * D) * LOG2PI
    total = (-0.5 * s_total + dot + w_term
             - N * (0.5 * D) * LOG2PI + prior)
    o_ref[...] = jnp.broadcast_to(total, (1, 1))


def kernel(xs, ids, means, mean_mean, bin_probs):
    ids32 = ids.astype(jnp.int32)
    xs4 = xs.T.reshape(2, 8, NB, 128).transpose(0, 2, 1, 3)
    g_p, c_p, s_p = _sc_pass(xs4, ids32)
    g_t, aux = _tc_pass(xs, ids32.reshape(N // TCB, 1, TCB))
    out = pl.pallas_call(
        _combine_body,
        out_shape=jax.ShapeDtypeStruct((1, 1), jnp.float32),
    )(g_p, c_p, s_p, g_t, aux,
      means, means.T.reshape(1, K * D),
      mean_mean.reshape(1, D), bin_probs.reshape(1, K))
    return out[0, 0]


# speedup vs baseline: 3.8671x; 3.8671x over previous
import math

import jax
import jax.numpy as jnp
from jax import lax
from jax.experimental import pallas as pl
from jax.experimental.pallas import tpu as pltpu
from jax.experimental.pallas import tpu_sc as plsc

N = 1048576
D = 16
K = 16
L = 16
NC = 2
NS = 16
NW = NC * NS
NSC = 655360
PTS_PER_W = NSC // NW
CHUNK = 2048
NCHUNK = PTS_PER_W // CHUNK
GROUPS = CHUNK // L
NB = N // 128
CB = CHUNK // 128
BPW = PTS_PER_W // 128
TCB = 4096
NTC = (N - NSC) // TCB
LOG2PI = math.log(2.0 * math.pi)


def _sc_body(xs_hbm, ids_hbm, g_out, c_out, s_out,
             xa, xb, ia, ib, gacc, cacc, gsm, csm, sqv,
             sxa, sxb, sia, sib):
    wid = lax.axis_index("s") * NC + lax.axis_index("c")
    pbase0 = wid * PTS_PER_W

    zero = jnp.zeros((L,), jnp.float32)
    for k in range(K * D):
        gacc[pl.ds(k * L, L)] = zero
    for k in range(K):
        cacc[pl.ds(k * L, L)] = zero

    iota = lax.iota(jnp.int32, L)
    ones = jnp.ones((L,), jnp.float32)

    def xcopy0(c, buf, sem):
        return pltpu.make_async_copy(
            xs_hbm.at[0, pl.ds(wid * BPW + c * CB, CB), :, :], buf.at[0], sem)

    def xcopy1(c, buf, sem):
        return pltpu.make_async_copy(
            xs_hbm.at[1, pl.ds(wid * BPW + c * CB, CB), :, :], buf.at[1], sem)

    def xcopy(c, buf, sem):
        class _Pair:
            def start(self):
                xcopy0(c, buf, sem).start()
                xcopy1(c, buf, sem).start()

            def wait(self):
                xcopy0(c, buf, sem).wait()
                xcopy1(c, buf, sem).wait()
        return _Pair()

    def icopy(c, buf, sem):
        return pltpu.make_async_copy(
            ids_hbm.at[pl.ds(pbase0 + c * CHUNK, CHUNK)], buf, sem)

    def process(xbuf, ibuf, sq):
        def grp(g, sq):
            sq0, sq1, sq2, sq3 = sq
            idv = ibuf[pl.ds(g * L, L)]
            pb = idv * L + iota
            plsc.addupdate_scatter(cacc, [pb], ones)
            jb = g // 8
            ii0 = (g % 8) * L
            xc = [xbuf[dt, jb, dr, pl.ds(ii0, L)]
                  for dt in range(2) for dr in range(8)]
            for d in range(D):
                plsc.addupdate_scatter(
                    gacc.at[pl.ds(d * K * L, K * L)], [pb], xc[d])
            for d in range(0, D, 4):
                sq0 = sq0 + xc[d] * xc[d]
                sq1 = sq1 + xc[d + 1] * xc[d + 1]
                sq2 = sq2 + xc[d + 2] * xc[d + 2]
                sq3 = sq3 + xc[d + 3] * xc[d + 3]
            return sq0, sq1, sq2, sq3
        return plsc.parallel_loop(0, GROUPS, 1, unroll=2, carry=sq)(grp)

    xcopy(0, xa, sxa).start()
    icopy(0, ia, sia).start()

    def outer(i, sq):
        ca = 2 * i
        cb = 2 * i + 1
        xcopy(ca, xa, sxa).wait()
        icopy(ca, ia, sia).wait()
        xcopy(cb, xb, sxb).start()
        icopy(cb, ib, sib).start()
        sq = process(xa, ia, sq)
        xcopy(cb, xb, sxb).wait()
        icopy(cb, ib, sib).wait()

        @pl.when(cb + 1 < NCHUNK)
        def _():
            xcopy(cb + 1, xa, sxa).start()
            icopy(cb + 1, ia, sia).start()

        sq = process(xb, ib, sq)
        return sq

    zacc = (jnp.zeros((L,), jnp.float32),) * 4
    sq = lax.fori_loop(0, NCHUNK // 2, outer, zacc)

    sqv[...] = sq[0] + sq[1] + sq[2] + sq[3]

    mask15 = iota == jnp.full((L,), L - 1, jnp.int32)
    for m in range(K * D):
        v = gacc[pl.ds(m * L, L)]
        plsc.store_scatter(gsm, [jnp.full((L,), m, jnp.int32)],
                           plsc.cumsum(v), mask=mask15)
    for m in range(K):
        v = cacc[pl.ds(m * L, L)]
        plsc.store_scatter(csm, [jnp.full((L,), m, jnp.int32)],
                           plsc.cumsum(v), mask=mask15)

    pltpu.sync_copy(gsm, g_out.at[wid])
    pltpu.sync_copy(csm, c_out.at[wid])
    pltpu.sync_copy(sqv, s_out.at[wid])


_sc_pass = pl.kernel(
    _sc_body,
    out_type=(
        jax.ShapeDtypeStruct((NW, K * D), jnp.float32),
        jax.ShapeDtypeStruct((NW, K), jnp.float32),
        jax.ShapeDtypeStruct((NW, L), jnp.float32),
    ),
    mesh=plsc.VectorSubcoreMesh(core_axis_name="c", subcore_axis_name="s"),
    compiler_params=pltpu.CompilerParams(
        needs_layout_passes=False, use_tc_tiling_on_sc=False),
    scratch_types=[
        pltpu.VMEM((2, CB, 8, 128), jnp.float32),
        pltpu.VMEM((2, CB, 8, 128), jnp.float32),
        pltpu.VMEM((CHUNK,), jnp.int32),
        pltpu.VMEM((CHUNK,), jnp.int32),
        pltpu.VMEM((L * K * D,), jnp.float32),
        pltpu.VMEM((L * K,), jnp.float32),
        pltpu.VMEM((K * D,), jnp.float32),
        pltpu.VMEM((K,), jnp.float32),
        pltpu.VMEM((L,), jnp.float32),
        pltpu.SemaphoreType.DMA,
        pltpu.SemaphoreType.DMA,
        pltpu.SemaphoreType.DMA,
        pltpu.SemaphoreType.DMA,
    ],
)


def _tc_body(x_ref, i_ref, g_ref, aux_ref):
    b = pl.program_id(0)

    @pl.when(b == 0)
    def _():
        g_ref[...] = jnp.zeros_like(g_ref)
        aux_ref[...] = jnp.zeros_like(aux_ref)

    xt = x_ref[...]
    ids = i_ref[0, 0, :]
    oh = (ids[:, None]
          == lax.broadcasted_iota(jnp.int32, (TCB, K), 1)
          ).astype(jnp.float32)
    g_ref[...] += lax.dot_general(
        xt, oh, (((1,), (0,)), ((), ())),
        preferred_element_type=jnp.float32)
    aux_ref[...] += jnp.stack(
        [jnp.sum(oh, axis=0), jnp.sum(xt * xt, axis=1)])


_tc_pass = pl.pallas_call(
    _tc_body,
    grid=(NTC,),
    in_specs=[
        pl.BlockSpec((D, TCB), lambda b: (0, b + NSC // TCB)),
        pl.BlockSpec((1, 1, TCB), lambda b: (b + NSC // TCB, 0, 0)),
    ],
    out_specs=[
        pl.BlockSpec((D, K), lambda b: (0, 0)),
        pl.BlockSpec((2, K), lambda b: (0, 0)),
    ],
    out_shape=[
        jax.ShapeDtypeStruct((D, K), jnp.float32),
        jax.ShapeDtypeStruct((2, K), jnp.float32),
    ],
)


def _combine_body(g_ref, c_ref, s_ref, gt_ref, aux_ref, mu_ref, mut_ref,
                  muf_ref, mm_ref, bp_ref, o_ref):
    g = jnp.sum(g_ref[...], axis=0)
    aux = aux_ref[...]
    cnt = jnp.sum(c_ref[...], axis=0) + aux[0]
    s_total = jnp.sum(s_ref[...]) + jnp.sum(aux[1])
    mu = mu_ref[...]
    musq = jnp.sum(mu * mu, axis=1)
    logbp = jnp.log(bp_ref[...])[0]
    dot = (jnp.sum(g * muf_ref[...][0])
           + jnp.sum(gt_ref[...] * mut_ref[...]))
    w_term = jnp.sum(cnt * (logbp - 0.5 * musq))
    pm = mu - mm_ref[...]
    prior = -0.5 * jnp.sum(pm * pm) - K * (0.5 * D) * LOG2PI
    total = (-0.5 * s_total + dot + w_term
             - N * (0.5 * D) * LOG2PI + prior)
    o_ref[...] = jnp.broadcast_to(total, (1, 1))


def kernel(xs, ids, means, mean_mean, bin_probs):
    ids32 = ids.astype(jnp.int32)
    xs4 = xs.T.reshape(2, 8, NB, 128).transpose(0, 2, 1, 3)
    g_p, c_p, s_p = _sc_pass(xs4, ids32)
    g_t, aux = _tc_pass(xs.T, ids32.reshape(N // TCB, 1, TCB))
    out = pl.pallas_call(
        _combine_body,
        out_shape=jax.ShapeDtypeStruct((1, 1), jnp.float32),
    )(g_p, c_p, s_p, g_t, aux,
      means, means.T, means.T.reshape(1, K * D),
      mean_mean.reshape(1, D), bin_probs.reshape(1, K))
    return out[0, 0]


# speedup vs baseline: 5.0175x; 1.2975x over previous
import math

import jax
import jax.numpy as jnp
from jax import lax
from jax.experimental import pallas as pl
from jax.experimental.pallas import tpu as pltpu
from jax.experimental.pallas import tpu_sc as plsc

N = 1048576
D = 16
K = 16
L = 16
NC = 2
NS = 16
NW = NC * NS
NSC = 655360
PTS_PER_W = NSC // NW
CHUNK = 2048
NCHUNK = PTS_PER_W // CHUNK
GROUPS = CHUNK // L
NB = N // 128
CB = CHUNK // 128
BPW = PTS_PER_W // 128
TCB = 4096
NTC = (N - NSC) // TCB
LOG2PI = math.log(2.0 * math.pi)


def _sc_body(xs_hbm, ids_hbm, g_out, c_out, s_out,
             xa, xb, ia, ib, gacc, cacc, gsm, csm, sqv,
             sxa, sxb, sia, sib):
    wid = lax.axis_index("s") * NC + lax.axis_index("c")
    pbase0 = wid * PTS_PER_W

    zero = jnp.zeros((L,), jnp.float32)
    for k in range(K * D):
        gacc[pl.ds(k * L, L)] = zero
    for k in range(K):
        cacc[pl.ds(k * L, L)] = zero

    iota = lax.iota(jnp.int32, L)
    ones = jnp.ones((L,), jnp.float32)

    def xcopy0(c, buf, sem):
        return pltpu.make_async_copy(
            xs_hbm.at[0, pl.ds(wid * BPW + c * CB, CB), :, :], buf.at[0], sem)

    def xcopy1(c, buf, sem):
        return pltpu.make_async_copy(
            xs_hbm.at[1, pl.ds(wid * BPW + c * CB, CB), :, :], buf.at[1], sem)

    def xcopy(c, buf, sem):
        class _Pair:
            def start(self):
                xcopy0(c, buf, sem).start()
                xcopy1(c, buf, sem).start()

            def wait(self):
                xcopy0(c, buf, sem).wait()
                xcopy1(c, buf, sem).wait()
        return _Pair()

    def icopy(c, buf, sem):
        return pltpu.make_async_copy(
            ids_hbm.at[pl.ds(pbase0 + c * CHUNK, CHUNK)], buf, sem)

    def process(xbuf, ibuf, sq):
        def grp(g, sq):
            sq0, sq1, sq2, sq3 = sq
            idv = ibuf[pl.ds(g * L, L)]
            pb = idv * L + iota
            plsc.addupdate_scatter(cacc, [pb], ones)
            jb = g // 8
            ii0 = (g % 8) * L
            xc = [xbuf[dt, jb, dr, pl.ds(ii0, L)]
                  for dt in range(2) for dr in range(8)]
            for d in range(D):
                plsc.addupdate_scatter(
                    gacc.at[pl.ds(d * K * L, K * L)], [pb], xc[d])
            for d in range(0, D, 4):
                sq0 = sq0 + xc[d] * xc[d]
                sq1 = sq1 + xc[d + 1] * xc[d + 1]
                sq2 = sq2 + xc[d + 2] * xc[d + 2]
                sq3 = sq3 + xc[d + 3] * xc[d + 3]
            return sq0, sq1, sq2, sq3
        return plsc.parallel_loop(0, GROUPS, 1, unroll=2, carry=sq)(grp)

    xcopy(0, xa, sxa).start()
    icopy(0, ia, sia).start()

    def outer(i, sq):
        ca = 2 * i
        cb = 2 * i + 1
        xcopy(ca, xa, sxa).wait()
        icopy(ca, ia, sia).wait()
        xcopy(cb, xb, sxb).start()
        icopy(cb, ib, sib).start()
        sq = process(xa, ia, sq)
        xcopy(cb, xb, sxb).wait()
        icopy(cb, ib, sib).wait()

        @pl.when(cb + 1 < NCHUNK)
        def _():
            xcopy(cb + 1, xa, sxa).start()
            icopy(cb + 1, ia, sia).start()

        sq = process(xb, ib, sq)
        return sq

    zacc = (jnp.zeros((L,), jnp.float32),) * 4
    sq = lax.fori_loop(0, NCHUNK // 2, outer, zacc)

    sqv[...] = sq[0] + sq[1] + sq[2] + sq[3]

    mask15 = iota == jnp.full((L,), L - 1, jnp.int32)
    for m in range(K * D):
        v = gacc[pl.ds(m * L, L)]
        plsc.store_scatter(gsm, [jnp.full((L,), m, jnp.int32)],
                           plsc.cumsum(v), mask=mask15)
    for m in range(K):
        v = cacc[pl.ds(m * L, L)]
        plsc.store_scatter(csm, [jnp.full((L,), m, jnp.int32)],
                           plsc.cumsum(v), mask=mask15)

    pltpu.sync_copy(gsm, g_out.at[wid])
    pltpu.sync_copy(csm, c_out.at[wid])
    pltpu.sync_copy(sqv, s_out.at[wid])


_sc_pass = pl.kernel(
    _sc_body,
    out_type=(
        jax.ShapeDtypeStruct((NW, K * D), jnp.float32),
        jax.ShapeDtypeStruct((NW, K), jnp.float32),
        jax.ShapeDtypeStruct((NW, L), jnp.float32),
    ),
    mesh=plsc.VectorSubcoreMesh(core_axis_name="c", subcore_axis_name="s"),
    compiler_params=pltpu.CompilerParams(
        needs_layout_passes=False, use_tc_tiling_on_sc=False),
    scratch_types=[
        pltpu.VMEM((2, CB, 8, 128), jnp.float32),
        pltpu.VMEM((2, CB, 8, 128), jnp.float32),
        pltpu.VMEM((CHUNK,), jnp.int32),
        pltpu.VMEM((CHUNK,), jnp.int32),
        pltpu.VMEM((L * K * D,), jnp.float32),
        pltpu.VMEM((L * K,), jnp.float32),
        pltpu.VMEM((K * D,), jnp.float32),
        pltpu.VMEM((K,), jnp.float32),
        pltpu.VMEM((L,), jnp.float32),
        pltpu.SemaphoreType.DMA,
        pltpu.SemaphoreType.DMA,
        pltpu.SemaphoreType.DMA,
        pltpu.SemaphoreType.DMA,
    ],
)


def _tc_body(x_ref, i_ref, g_ref, aux_ref):
    b = pl.program_id(0)

    @pl.when(b == 0)
    def _():
        g_ref[...] = jnp.zeros_like(g_ref)
        aux_ref[...] = jnp.zeros_like(aux_ref)

    xt = x_ref[...]
    ids = i_ref[0, 0, :]
    oht = (lax.broadcasted_iota(jnp.int32, (K, TCB), 0)
           == ids[None, :]).astype(jnp.float32)
    g_ref[...] += lax.dot_general(
        xt, oht, (((1,), (1,)), ((), ())),
        preferred_element_type=jnp.float32)
    aux_ref[...] += jnp.stack(
        [jnp.sum(oht, axis=1), jnp.sum(xt * xt, axis=1)])


_tc_pass = pl.pallas_call(
    _tc_body,
    grid=(NTC,),
    in_specs=[
        pl.BlockSpec((D, TCB), lambda b: (0, b + NSC // TCB)),
        pl.BlockSpec((1, 1, TCB), lambda b: (b + NSC // TCB, 0, 0)),
    ],
    out_specs=[
        pl.BlockSpec((D, K), lambda b: (0, 0)),
        pl.BlockSpec((2, K), lambda b: (0, 0)),
    ],
    out_shape=[
        jax.ShapeDtypeStruct((D, K), jnp.float32),
        jax.ShapeDtypeStruct((2, K), jnp.float32),
    ],
)


def _combine_body(g_ref, c_ref, s_ref, gt_ref, aux_ref, mu_ref, mut_ref,
                  muf_ref, mm_ref, bp_ref, o_ref):
    g = jnp.sum(g_ref[...], axis=0)
    aux = aux_ref[...]
    cnt = jnp.sum(c_ref[...], axis=0) + aux[0]
    s_total = jnp.sum(s_ref[...]) + jnp.sum(aux[1])
    mu = mu_ref[...]
    musq = jnp.sum(mu * mu, axis=1)
    logbp = jnp.log(bp_ref[...])[0]
    dot = (jnp.sum(g * muf_ref[...][0])
           + jnp.sum(gt_ref[...] * mut_ref[...]))
    w_term = jnp.sum(cnt * (logbp - 0.5 * musq))
    pm = mu - mm_ref[...]
    prior = -0.5 * jnp.sum(pm * pm) - K * (0.5 * D) * LOG2PI
    total = (-0.5 * s_total + dot + w_term
             - N * (0.5 * D) * LOG2PI + prior)
    o_ref[...] = jnp.broadcast_to(total, (1, 1))


def kernel(xs, ids, means, mean_mean, bin_probs):
    ids32 = ids.astype(jnp.int32)
    xs4 = xs.T.reshape(2, 8, NB, 128).transpose(0, 2, 1, 3)
    g_p, c_p, s_p = _sc_pass(xs4, ids32)
    g_t, aux = _tc_pass(xs.T, ids32.reshape(N // TCB, 1, TCB))
    out = pl.pallas_call(
        _combine_body,
        out_shape=jax.ShapeDtypeStruct((1, 1), jnp.float32),
    )(g_p, c_p, s_p, g_t, aux,
      means, means.T, means.T.reshape(1, K * D),
      mean_mean.reshape(1, D), bin_probs.reshape(1, K))
    return out[0, 0]


# speedup vs baseline: 6.0353x; 1.2028x over previous
import math

import jax
import jax.numpy as jnp
from jax import lax
from jax.experimental import pallas as pl
from jax.experimental.pallas import tpu as pltpu
from jax.experimental.pallas import tpu_sc as plsc

N = 1048576
D = 16
K = 16
L = 16
NC = 2
NS = 16
NW = NC * NS
NSC = 786432
PTS_PER_W = NSC // NW
CHUNK = 2048
NCHUNK = PTS_PER_W // CHUNK
GROUPS = CHUNK // L
NB = N // 128
CB = CHUNK // 128
BPW = PTS_PER_W // 128
TCB = 4096
NTC = (N - NSC) // TCB
LOG2PI = math.log(2.0 * math.pi)


def _sc_body(xs_hbm, ids_hbm, g_out, c_out, s_out,
             xa, xb, ia, ib, gacc, cacc, gsm, csm, sqv,
             sxa, sxb, sia, sib):
    wid = lax.axis_index("s") * NC + lax.axis_index("c")
    pbase0 = wid * PTS_PER_W

    zero = jnp.zeros((L,), jnp.float32)
    for k in range(K * D):
        gacc[pl.ds(k * L, L)] = zero
    for k in range(K):
        cacc[pl.ds(k * L, L)] = zero

    iota = lax.iota(jnp.int32, L)
    ones = jnp.ones((L,), jnp.float32)

    def xcopy0(c, buf, sem):
        return pltpu.make_async_copy(
            xs_hbm.at[0, pl.ds(wid * BPW + c * CB, CB), :, :], buf.at[0], sem)

    def xcopy1(c, buf, sem):
        return pltpu.make_async_copy(
            xs_hbm.at[1, pl.ds(wid * BPW + c * CB, CB), :, :], buf.at[1], sem)

    def xcopy(c, buf, sem):
        class _Pair:
            def start(self):
                xcopy0(c, buf, sem).start()
                xcopy1(c, buf, sem).start()

            def wait(self):
                xcopy0(c, buf, sem).wait()
                xcopy1(c, buf, sem).wait()
        return _Pair()

    def icopy(c, buf, sem):
        return pltpu.make_async_copy(
            ids_hbm.at[pl.ds(pbase0 + c * CHUNK, CHUNK)], buf, sem)

    def process(xbuf, ibuf, sq):
        def grp(g, sq):
            sq0, sq1, sq2, sq3 = sq
            idv = ibuf[pl.ds(g * L, L)]
            pb = idv * L + iota
            plsc.addupdate_scatter(cacc, [pb], ones)
            jb = g // 8
            ii0 = (g % 8) * L
            xc = [xbuf[dt, jb, dr, pl.ds(ii0, L)]
                  for dt in range(2) for dr in range(8)]
            for d in range(D):
                plsc.addupdate_scatter(
                    gacc.at[pl.ds(d * K * L, K * L)], [pb], xc[d])
            for d in range(0, D, 4):
                sq0 = sq0 + xc[d] * xc[d]
                sq1 = sq1 + xc[d + 1] * xc[d + 1]
                sq2 = sq2 + xc[d + 2] * xc[d + 2]
                sq3 = sq3 + xc[d + 3] * xc[d + 3]
            return sq0, sq1, sq2, sq3
        return plsc.parallel_loop(0, GROUPS, 1, unroll=2, carry=sq)(grp)

    xcopy(0, xa, sxa).start()
    icopy(0, ia, sia).start()

    def outer(i, sq):
        ca = 2 * i
        cb = 2 * i + 1
        xcopy(ca, xa, sxa).wait()
        icopy(ca, ia, sia).wait()
        xcopy(cb, xb, sxb).start()
        icopy(cb, ib, sib).start()
        sq = process(xa, ia, sq)
        xcopy(cb, xb, sxb).wait()
        icopy(cb, ib, sib).wait()

        @pl.when(cb + 1 < NCHUNK)
        def _():
            xcopy(cb + 1, xa, sxa).start()
            icopy(cb + 1, ia, sia).start()

        sq = process(xb, ib, sq)
        return sq

    zacc = (jnp.zeros((L,), jnp.float32),) * 4
    sq = lax.fori_loop(0, NCHUNK // 2, outer, zacc)

    sqv[...] = sq[0] + sq[1] + sq[2] + sq[3]

    mask15 = iota == jnp.full((L,), L - 1, jnp.int32)
    for m in range(K * D):
        v = gacc[pl.ds(m * L, L)]
        plsc.store_scatter(gsm, [jnp.full((L,), m, jnp.int32)],
                           plsc.cumsum(v), mask=mask15)
    for m in range(K):
        v = cacc[pl.ds(m * L, L)]
        plsc.store_scatter(csm, [jnp.full((L,), m, jnp.int32)],
                           plsc.cumsum(v), mask=mask15)

    pltpu.sync_copy(gsm, g_out.at[wid])
    pltpu.sync_copy(csm, c_out.at[wid])
    pltpu.sync_copy(sqv, s_out.at[wid])


_sc_pass = pl.kernel(
    _sc_body,
    out_type=(
        jax.ShapeDtypeStruct((NW, K * D), jnp.float32),
        jax.ShapeDtypeStruct((NW, K), jnp.float32),
        jax.ShapeDtypeStruct((NW, L), jnp.float32),
    ),
    mesh=plsc.VectorSubcoreMesh(core_axis_name="c", subcore_axis_name="s"),
    compiler_params=pltpu.CompilerParams(
        needs_layout_passes=False, use_tc_tiling_on_sc=False),
    scratch_types=[
        pltpu.VMEM((2, CB, 8, 128), jnp.float32),
        pltpu.VMEM((2, CB, 8, 128), jnp.float32),
        pltpu.VMEM((CHUNK,), jnp.int32),
        pltpu.VMEM((CHUNK,), jnp.int32),
        pltpu.VMEM((L * K * D,), jnp.float32),
        pltpu.VMEM((L * K,), jnp.float32),
        pltpu.VMEM((K * D,), jnp.float32),
        pltpu.VMEM((K,), jnp.float32),
        pltpu.VMEM((L,), jnp.float32),
        pltpu.SemaphoreType.DMA,
        pltpu.SemaphoreType.DMA,
        pltpu.SemaphoreType.DMA,
        pltpu.SemaphoreType.DMA,
    ],
)


def _tc_body(x_ref, i_ref, g_ref, aux_ref):
    b = pl.program_id(0)

    @pl.when(b == 0)
    def _():
        g_ref[...] = jnp.zeros_like(g_ref)
        aux_ref[...] = jnp.zeros_like(aux_ref)

    xt = x_ref[...]
    ids = i_ref[0, 0, :]
    oht = (lax.broadcasted_iota(jnp.int32, (K, TCB), 0)
           == ids[None, :]).astype(jnp.float32)
    g_ref[...] += lax.dot_general(
        xt, oht, (((1,), (1,)), ((), ())),
        preferred_element_type=jnp.float32)
    aux_ref[...] += jnp.stack(
        [jnp.sum(oht, axis=1), jnp.sum(xt * xt, axis=1)])


_tc_pass = pl.pallas_call(
    _tc_body,
    grid=(NTC,),
    in_specs=[
        pl.BlockSpec((D, TCB), lambda b: (0, b + NSC // TCB)),
        pl.BlockSpec((1, 1, TCB), lambda b: (b + NSC // TCB, 0, 0)),
    ],
    out_specs=[
        pl.BlockSpec((D, K), lambda b: (0, 0)),
        pl.BlockSpec((2, K), lambda b: (0, 0)),
    ],
    out_shape=[
        jax.ShapeDtypeStruct((D, K), jnp.float32),
        jax.ShapeDtypeStruct((2, K), jnp.float32),
    ],
)


def _combine_body(g_ref, c_ref, s_ref, gt_ref, aux_ref, mu_ref, mut_ref,
                  muf_ref, mm_ref, bp_ref, o_ref):
    g = jnp.sum(g_ref[...], axis=0)
    aux = aux_ref[...]
    cnt = jnp.sum(c_ref[...], axis=0) + aux[0]
    s_total = jnp.sum(s_ref[...]) + jnp.sum(aux[1])
    mu = mu_ref[...]
    musq = jnp.sum(mu * mu, axis=1)
    logbp = jnp.log(bp_ref[...])[0]
    dot = (jnp.sum(g * muf_ref[...][0])
           + jnp.sum(gt_ref[...] * mut_ref[...]))
    w_term = jnp.sum(cnt * (logbp - 0.5 * musq))
    pm = mu - mm_ref[...]
    prior = -0.5 * jnp.sum(pm * pm) - K * (0.5 * D) * LOG2PI
    total = (-0.5 * s_total + dot + w_term
             - N * (0.5 * D) * LOG2PI + prior)
    o_ref[...] = jnp.broadcast_to(total, (1, 1))


def kernel(xs, ids, means, mean_mean, bin_probs):
    ids32 = ids.astype(jnp.int32)
    xs4 = xs.T.reshape(2, 8, NB, 128).transpose(0, 2, 1, 3)
    g_p, c_p, s_p = _sc_pass(xs4, ids32)
    g_t, aux = _tc_pass(xs.T, ids32.reshape(N // TCB, 1, TCB))
    out = pl.pallas_call(
        _combine_body,
        out_shape=jax.ShapeDtypeStruct((1, 1), jnp.float32),
    )(g_p, c_p, s_p, g_t, aux,
      means, means.T, means.T.reshape(1, K * D),
      mean_mean.reshape(1, D), bin_probs.reshape(1, K))
    return out[0, 0]
